# baseline (speedup 1.0000x reference)
import functools

import jax
import jax.numpy as jnp
from jax import lax
from jax.experimental import pallas as pl
from jax.experimental.pallas import tpu as pltpu
from jax.experimental.pallas import tpu_sc as plsc

_PI = 3.141592653589793

_SPOLY = (-9.99449986e-01, 1.65838221e-01, -7.99852030e-03, 1.47736456e-04)
_CPOLY = (-9.99971081e-01, 4.99837540e-01, -4.15222679e-02, 1.34409944e-03,
          -1.90647593e-05)

_NW = 32


def _poly(z, coeffs):
    acc = jnp.full((16,), coeffs[-1], dtype=jnp.float32)
    for c in reversed(coeffs[:-1]):
        acc = acc * z + c
    return acc


_W = 512


def _make_detile_kernel(v: int, d: int):
    nb = v // _W
    tail = v - nb * _W
    assert tail < 128, "tail must fit one pre-sliced staging buffer"
    per_w = nb // _NW
    extra = nb - per_w * _NW
    mesh = plsc.VectorSubcoreMesh(core_axis_name="c", subcore_axis_name="s")
    bsz = _W * d

    @functools.partial(
        pl.kernel,
        out_type=(jax.ShapeDtypeStruct((v * d,), jnp.float32),
                  jax.ShapeDtypeStruct((v * d,), jnp.float32)),
        mesh=mesh,
        compiler_params=pltpu.CompilerParams(use_tc_tiling_on_sc=True,
                                             needs_layout_passes=False),
        scratch_types=[
            pltpu.VMEM((d, _W), jnp.float32),
            pltpu.VMEM((d, _W), jnp.float32),
            pltpu.VMEM((bsz,), jnp.float32),
            pltpu.VMEM((bsz,), jnp.float32),
            pltpu.SemaphoreType.DMA,
            pltpu.SemaphoreType.DMA,
            pltpu.SemaphoreType.DMA,
            pltpu.SemaphoreType.DMA,
        ],
    )
    def detile(rho_hbm, theta_hbm, rtail_hbm, ttail_hbm, rout_hbm, tout_hbm,
               buf0, buf1, tbuf0, tbuf1, in0, in1, wo0, wo1):
        wid = lax.axis_index("s") * 2 + lax.axis_index("c")
        n_mine = per_w + jnp.where(wid < extra, 1, 0)
        iot = lax.iota(jnp.int32, 16)

        def run_table(tab_hbm, out_hbm):
            bufs = (buf0, buf1)
            tbufs = (tbuf0, tbuf1)
            isems = (in0, in1)
            wsems = (wo0, wo1)

            def read(i, slot):
                c = wid + i * _NW
                pltpu.async_copy(
                    tab_hbm.at[:, pl.ds(c * _W, _W)], bufs[slot],
                    isems[slot])

            def wait_read(slot):
                pltpu.make_async_copy(
                    tab_hbm.at[:, pl.ds(0, _W)], bufs[slot],
                    isems[slot]).wait()

            iot_d = iot * d

            def transpose(slot):
                buf = bufs[slot]
                tbuf = tbufs[slot]

                @plsc.parallel_loop(0, d, unroll=2)
                def _(k):
                    for g in range(_W // 16):
                        vec = buf[k, pl.ds(g * 16, 16)]
                        scat = iot_d + (g * 16 * d + k)
                        plsc.store_scatter(tbuf, [scat], vec)

            def write(i, slot):
                c = wid + i * _NW
                pltpu.async_copy(
                    tbufs[slot], out_hbm.at[pl.ds(c * bsz, bsz)], wsems[slot])

            def wait_write(slot):
                pltpu.make_async_copy(
                    tbufs[slot], out_hbm.at[pl.ds(0, bsz)], wsems[slot]).wait()

            @pl.when(n_mine > 0)
            def _():
                read(0, 0)

            npairs = (per_w + 2) // 2

            def pair_body(g, _):
                for s in range(2):
                    i = g * 2 + s

                    @pl.when(i < n_mine)
                    def _():
                        @pl.when(i + 1 < n_mine)
                        def _():
                            read(i + 1, 1 - s)

                        wait_read(s)

                        @pl.when(i >= 2)
                        def _():
                            wait_write(s)

                        transpose(s)
                        write(i, s)
                return ()

            lax.fori_loop(0, npairs, pair_body, ())

            @pl.when(n_mine > 0)
            def _():
                wait_write(0)

            @pl.when(n_mine > 1)
            def _():
                wait_write(1)

        run_table(rho_hbm, rout_hbm)
        run_table(theta_hbm, tout_hbm)

        if tail:
            @pl.when(wid == _NW - 1)
            def _():
                pltpu.sync_copy(rtail_hbm, tbuf0.at[pl.ds(0, tail * d)])
                pltpu.sync_copy(tbuf0.at[pl.ds(0, tail * d)],
                                rout_hbm.at[pl.ds(nb * bsz, tail * d)])

            @pl.when(wid == _NW - 2)
            def _():
                pltpu.sync_copy(ttail_hbm, tbuf1.at[pl.ds(0, tail * d)])
                pltpu.sync_copy(tbuf1.at[pl.ds(0, tail * d)],
                                tout_hbm.at[pl.ds(nb * bsz, tail * d)])

    return detile


def _make_lookup_kernel(t_dim: int, b_dim: int, v: int, d: int):
    n_blk = b_dim // 128
    n_units = t_dim * n_blk
    per_w = n_units // _NW
    mesh = plsc.VectorSubcoreMesh(core_axis_name="c", subcore_axis_name="s")

    @functools.partial(
        pl.kernel,
        out_type=jax.ShapeDtypeStruct((t_dim, 2 * d, b_dim), jnp.float32),
        mesh=mesh,
        compiler_params=pltpu.CompilerParams(needs_layout_passes=False,
                                             use_tc_tiling_on_sc=False),
        scratch_types=[
            pltpu.VMEM((128,), jnp.int32),
            pltpu.VMEM((128,), jnp.int32),
            pltpu.VMEM((128, d), jnp.float32),
            pltpu.VMEM((128, d), jnp.float32),
            pltpu.VMEM((128, d), jnp.float32),
            pltpu.VMEM((128, d), jnp.float32),
            pltpu.VMEM((2 * d, 128), jnp.float32),
            pltpu.VMEM((2 * d, 128), jnp.float32),
            pltpu.SemaphoreType.DMA,
            pltpu.SemaphoreType.DMA,
            pltpu.SemaphoreType.DMA,
            pltpu.SemaphoreType.DMA,
        ],
    )
    def lookup(idx_hbm, rho_hbm, theta_hbm, out_hbm,
               idx0, idx1, rr0, rr1, tt0, tt1, ov0, ov1,
               gs0, gs1, os0, os1):
        wid = lax.axis_index("s") * 2 + lax.axis_index("c")
        iot = lax.iota(jnp.int32, 16)

        idxs = (idx0, idx1)
        rrs = (rr0, rr1)
        tts = (tt0, tt1)
        ovs = (ov0, ov1)
        gsems = (gs0, gs1)
        osems = (os0, os1)

        def unit_id(i):
            return wid + i * _NW

        def stage(i, slot):
            u = unit_id(i)
            ti = u // n_blk
            blk = u % n_blk
            pltpu.sync_copy(idx_hbm.at[ti, pl.ds(blk * 128, 128)], idxs[slot])
            pltpu.async_copy(rho_hbm.at[idxs[slot]], rrs[slot], gsems[slot])
            pltpu.async_copy(theta_hbm.at[idxs[slot]], tts[slot], gsems[slot])

        def wait_gathers(slot):
            pltpu.make_async_copy(
                rho_hbm.at[pl.ds(0, 128), :], rrs[slot], gsems[slot]).wait()
            pltpu.make_async_copy(
                rho_hbm.at[pl.ds(0, 128), :], tts[slot], gsems[slot]).wait()

        def compute(slot):
            rows_r = rrs[slot]
            rows_t = tts[slot]
            out_v = ovs[slot]

            @plsc.parallel_loop(0, 8, unroll=4)
            def _(h):
                base = iot + h * 16
                for k in range(d):
                    kv = jnp.full((16,), k, jnp.int32)
                    r = plsc.load_gather(rows_r, [base, kv])
                    tv = plsc.load_gather(rows_t, [base, kv])
                    x = tv - jnp.float32(_PI)
                    z = x * x
                    rc = r * _poly(z, _CPOLY)
                    rs = (r * x) * _poly(z, _SPOLY)
                    out_v[2 * k, pl.ds(h * 16, 16)] = rc
                    out_v[2 * k + 1, pl.ds(h * 16, 16)] = rs

        def write(i, slot):
            u = unit_id(i)
            ti = u // n_blk
            blk = u % n_blk
            pltpu.async_copy(
                ovs[slot], out_hbm.at[ti, :, pl.ds(blk * 128, 128)],
                osems[slot])

        def drain_write(slot):
            pltpu.make_async_copy(
                ovs[slot], out_hbm.at[0, :, pl.ds(0, 128)],
                osems[slot]).wait()

        stage(0, 0)

        def pair_body(g, _):
            for s in range(2):
                i = g * 2 + s
                nxt = i + 1
                nslot = (s + 1) % 2

                @pl.when(nxt < per_w)
                def _():
                    stage(nxt, nslot)

                wait_gathers(s)

                @pl.when(i >= 2)
                def _():
                    drain_write(s)

                compute(s)
                write(i, s)
            return ()

        lax.fori_loop(0, per_w // 2, pair_body, ())
        drain_write(0)
        drain_write(1)

    return lookup


def kernel(idx, rho, theta):
    b_dim, t_dim = idx.shape
    v, d = rho.shape
    nb = v // _W
    tail = v - nb * _W

    idx_t = idx.T
    rho_t = rho.T
    theta_t = theta.T

    if tail:
        rtail = lax.slice(rho, (nb * _W, 0), (v, d)).reshape(tail * d)
        ttail = lax.slice(theta, (nb * _W, 0), (v, d)).reshape(tail * d)
    else:
        rtail = jnp.zeros((0,), jnp.float32)
        ttail = jnp.zeros((0,), jnp.float32)

    k1 = _make_detile_kernel(v, d)
    rflat, tflat = k1(rho_t, theta_t, rtail, ttail)
    rho_lin = rflat.reshape(v, d)
    theta_lin = tflat.reshape(v, d)

    k2 = _make_lookup_kernel(t_dim, b_dim, v, d)
    out3 = k2(idx_t, rho_lin, theta_lin)
    return out3.transpose(2, 0, 1)


# speedup vs baseline: 2.6415x; 2.6415x over previous
import functools

import jax
import jax.numpy as jnp
from jax import lax
from jax.experimental import pallas as pl
from jax.experimental.pallas import tpu as pltpu
from jax.experimental.pallas import tpu_sc as plsc

_PI = 3.141592653589793

_SPOLY = (-9.99449986e-01, 1.65838221e-01, -7.99852030e-03, 1.47736456e-04)
_CPOLY = (-9.99971081e-01, 4.99837540e-01, -4.15222679e-02, 1.34409944e-03,
          -1.90647593e-05)

_NW = 32


def _poly(z, coeffs):
    acc = jnp.full((16,), coeffs[-1], dtype=jnp.float32)
    for c in reversed(coeffs[:-1]):
        acc = acc * z + c
    return acc


_W = 512


def _make_detile_kernel(v: int, d: int):
    nb = v // _W
    tail = v - nb * _W
    assert tail < 128, "tail must fit one pre-sliced staging buffer"
    per_w = nb // _NW
    extra = nb - per_w * _NW
    mesh = plsc.VectorSubcoreMesh(core_axis_name="c", subcore_axis_name="s")
    bsz = _W * d

    @functools.partial(
        pl.kernel,
        out_type=(jax.ShapeDtypeStruct((v * d,), jnp.float32),
                  jax.ShapeDtypeStruct((v * d,), jnp.float32)),
        mesh=mesh,
        compiler_params=pltpu.CompilerParams(use_tc_tiling_on_sc=True,
                                             needs_layout_passes=False),
        scratch_types=[
            pltpu.VMEM((d, _W), jnp.float32),
            pltpu.VMEM((d, _W), jnp.float32),
            pltpu.VMEM((bsz,), jnp.float32),
            pltpu.VMEM((bsz,), jnp.float32),
            pltpu.SemaphoreType.DMA,
            pltpu.SemaphoreType.DMA,
            pltpu.SemaphoreType.DMA,
            pltpu.SemaphoreType.DMA,
        ],
    )
    def detile(rho_hbm, theta_hbm, rtail_hbm, ttail_hbm, rout_hbm, tout_hbm,
               buf0, buf1, tbuf0, tbuf1, in0, in1, wo0, wo1):
        wid = lax.axis_index("s") * 2 + lax.axis_index("c")
        n_mine = per_w + jnp.where(wid < extra, 1, 0)
        iot = lax.iota(jnp.int32, 16)

        def run_table(tab_hbm, out_hbm):
            bufs = (buf0, buf1)
            tbufs = (tbuf0, tbuf1)
            isems = (in0, in1)
            wsems = (wo0, wo1)

            def read(i, slot):
                c = wid + i * _NW
                pltpu.async_copy(
                    tab_hbm.at[:, pl.ds(c * _W, _W)], bufs[slot],
                    isems[slot])

            def wait_read(slot):
                pltpu.make_async_copy(
                    tab_hbm.at[:, pl.ds(0, _W)], bufs[slot],
                    isems[slot]).wait()

            iot_d = iot * d

            def transpose(slot):
                buf = bufs[slot]
                tbuf = tbufs[slot]

                @plsc.parallel_loop(0, d, unroll=2)
                def _(k):
                    for g in range(_W // 16):
                        vec = buf[k, pl.ds(g * 16, 16)]
                        rot = (iot + (g * 16 + k)) & (d - 1)
                        scat = (iot_d + (g * 16 * d)) + rot
                        plsc.store_scatter(tbuf, [scat], vec)

            def write(i, slot):
                c = wid + i * _NW
                pltpu.async_copy(
                    tbufs[slot], out_hbm.at[pl.ds(c * bsz, bsz)], wsems[slot])

            def wait_write(slot):
                pltpu.make_async_copy(
                    tbufs[slot], out_hbm.at[pl.ds(0, bsz)], wsems[slot]).wait()

            @pl.when(n_mine > 0)
            def _():
                read(0, 0)

            npairs = (per_w + 2) // 2

            def pair_body(g, _):
                for s in range(2):
                    i = g * 2 + s

                    @pl.when(i < n_mine)
                    def _():
                        @pl.when(i + 1 < n_mine)
                        def _():
                            read(i + 1, 1 - s)

                        wait_read(s)

                        @pl.when(i >= 2)
                        def _():
                            wait_write(s)

                        transpose(s)
                        write(i, s)
                return ()

            lax.fori_loop(0, npairs, pair_body, ())

            @pl.when(n_mine > 0)
            def _():
                wait_write(0)

            @pl.when(n_mine > 1)
            def _():
                wait_write(1)

        run_table(rho_hbm, rout_hbm)
        run_table(theta_hbm, tout_hbm)

        if tail:
            def skew_tail(tin_hbm, tout2_hbm, tb):
                n = tail * d
                pltpu.sync_copy(tin_hbm, tb.at[pl.ds(0, n)])

                def row(i, _):
                    for h in range(d // 16):
                        vec = tb[pl.ds(i * d + h * 16, 16)]
                        rot = (iot + (h * 16 + i)) & (d - 1)
                        plsc.store_scatter(tb, [rot + (n + i * d)], vec)
                    return ()

                lax.fori_loop(0, tail, row, ())
                pltpu.sync_copy(tb.at[pl.ds(n, n)],
                                tout2_hbm.at[pl.ds(nb * bsz, n)])

            @pl.when(wid == _NW - 1)
            def _():
                skew_tail(rtail_hbm, rout_hbm, tbuf0)

            @pl.when(wid == _NW - 2)
            def _():
                skew_tail(ttail_hbm, tout_hbm, tbuf1)

    return detile


def _make_lookup_kernel(t_dim: int, b_dim: int, v: int, d: int):
    n_blk = b_dim // 128
    n_units = t_dim * n_blk
    per_w = n_units // _NW
    mesh = plsc.VectorSubcoreMesh(core_axis_name="c", subcore_axis_name="s")

    @functools.partial(
        pl.kernel,
        out_type=jax.ShapeDtypeStruct((t_dim, 2 * d, b_dim), jnp.float32),
        mesh=mesh,
        compiler_params=pltpu.CompilerParams(needs_layout_passes=False,
                                             use_tc_tiling_on_sc=False),
        scratch_types=[
            pltpu.VMEM((128,), jnp.int32),
            pltpu.VMEM((128,), jnp.int32),
            pltpu.VMEM((128, d), jnp.float32),
            pltpu.VMEM((128, d), jnp.float32),
            pltpu.VMEM((128, d), jnp.float32),
            pltpu.VMEM((128, d), jnp.float32),
            pltpu.VMEM((2 * d, 128), jnp.float32),
            pltpu.VMEM((2 * d, 128), jnp.float32),
            pltpu.SemaphoreType.DMA,
            pltpu.SemaphoreType.DMA,
            pltpu.SemaphoreType.DMA,
            pltpu.SemaphoreType.DMA,
        ],
    )
    def lookup(idx_hbm, rho_hbm, theta_hbm, out_hbm,
               idx0, idx1, rr0, rr1, tt0, tt1, ov0, ov1,
               gs0, gs1, os0, os1):
        wid = lax.axis_index("s") * 2 + lax.axis_index("c")
        iot = lax.iota(jnp.int32, 16)

        idxs = (idx0, idx1)
        rrs = (rr0, rr1)
        tts = (tt0, tt1)
        ovs = (ov0, ov1)
        gsems = (gs0, gs1)
        osems = (os0, os1)

        def unit_id(i):
            return wid + i * _NW

        def stage(i, slot):
            u = unit_id(i)
            ti = u // n_blk
            blk = u % n_blk
            pltpu.sync_copy(idx_hbm.at[ti, pl.ds(blk * 128, 128)], idxs[slot])
            pltpu.async_copy(rho_hbm.at[idxs[slot]], rrs[slot], gsems[slot])
            pltpu.async_copy(theta_hbm.at[idxs[slot]], tts[slot], gsems[slot])

        def wait_gathers(slot):
            pltpu.make_async_copy(
                rho_hbm.at[pl.ds(0, 128), :], rrs[slot], gsems[slot]).wait()
            pltpu.make_async_copy(
                rho_hbm.at[pl.ds(0, 128), :], tts[slot], gsems[slot]).wait()

        def compute(slot):
            rows_r = rrs[slot]
            rows_t = tts[slot]
            out_v = ovs[slot]

            idx_ref = idxs[slot]

            @plsc.parallel_loop(0, 8, unroll=4)
            def _(h):
                base = iot + h * 16
                mrot = idx_ref[pl.ds(h * 16, 16)] & (d - 1)
                for k in range(d):
                    kv = (mrot + k) & (d - 1)
                    r = plsc.load_gather(rows_r, [base, kv])
                    tv = plsc.load_gather(rows_t, [base, kv])
                    x = tv - jnp.float32(_PI)
                    z = x * x
                    rc = r * _poly(z, _CPOLY)
                    rs = (r * x) * _poly(z, _SPOLY)
                    out_v[2 * k, pl.ds(h * 16, 16)] = rc
                    out_v[2 * k + 1, pl.ds(h * 16, 16)] = rs

        def write(i, slot):
            u = unit_id(i)
            ti = u // n_blk
            blk = u % n_blk
            pltpu.async_copy(
                ovs[slot], out_hbm.at[ti, :, pl.ds(blk * 128, 128)],
                osems[slot])

        def drain_write(slot):
            pltpu.make_async_copy(
                ovs[slot], out_hbm.at[0, :, pl.ds(0, 128)],
                osems[slot]).wait()

        stage(0, 0)

        def pair_body(g, _):
            for s in range(2):
                i = g * 2 + s
                nxt = i + 1
                nslot = (s + 1) % 2

                @pl.when(nxt < per_w)
                def _():
                    stage(nxt, nslot)

                wait_gathers(s)

                @pl.when(i >= 2)
                def _():
                    drain_write(s)

                compute(s)
                write(i, s)
            return ()

        lax.fori_loop(0, per_w // 2, pair_body, ())
        drain_write(0)
        drain_write(1)

    return lookup


def kernel(idx, rho, theta):
    b_dim, t_dim = idx.shape
    v, d = rho.shape
    nb = v // _W
    tail = v - nb * _W

    idx_t = idx.T
    rho_t = rho.T
    theta_t = theta.T

    if tail:
        rtail = lax.slice(rho, (nb * _W, 0), (v, d)).reshape(tail * d)
        ttail = lax.slice(theta, (nb * _W, 0), (v, d)).reshape(tail * d)
    else:
        rtail = jnp.zeros((0,), jnp.float32)
        ttail = jnp.zeros((0,), jnp.float32)

    k1 = _make_detile_kernel(v, d)
    rflat, tflat = k1(rho_t, theta_t, rtail, ttail)
    rho_lin = rflat.reshape(v, d)
    theta_lin = tflat.reshape(v, d)

    k2 = _make_lookup_kernel(t_dim, b_dim, v, d)
    out3 = k2(idx_t, rho_lin, theta_lin)
    return out3.transpose(2, 0, 1)
